# baseline (speedup 1.0000x reference)
import jax
import jax.numpy as jnp
from jax import lax
from jax.experimental import pallas as pl
from jax.experimental.pallas import tpu as pltpu

_K = 7
_B = 128
_NBLK = 1024 // _B


def _psroi_kernel(sp_ref, x_ref, o_ref, acc_ref):
    c = pl.program_id(0)
    b = pl.program_id(1)
    a = pl.program_id(2)
    i = c // _K
    j = c % _K

    rb = jnp.minimum(sp_ref[0, i] + a, _NBLK - 1)
    cb = jnp.minimum(sp_ref[1, j] + b, _NBLK - 1)
    ri0, ri1 = sp_ref[2, i], sp_ref[3, i]
    rj0, rj1 = sp_ref[4, j], sp_ref[5, j]

    rows = rb * _B + lax.broadcasted_iota(jnp.int32, (_B, _B), 0)
    row_ok = (rows >= ri0) & (rows < ri1) & ((a == 0) | (sp_ref[0, i] + a <= _NBLK - 1))
    xb = x_ref[0].astype(jnp.bfloat16).astype(jnp.float32)
    colpart = jnp.sum(jnp.where(row_ok, xb, 0.0), axis=0, keepdims=True)

    @pl.when(a == 0)
    def _stash():
        acc_ref[0:1, :] = colpart

    total = acc_ref[0:1, :] + colpart
    q = total.astype(jnp.bfloat16).astype(jnp.float32)
    cols = cb * _B + lax.broadcasted_iota(jnp.int32, (1, _B), 1)
    col_ok = (cols >= rj0) & (cols < rj1) & ((b == 0) | (sp_ref[1, j] + b <= _NBLK - 1))
    s = jnp.sum(jnp.where(col_ok, q, 0.0))
    part = jnp.full((1, 8, 128), s, dtype=jnp.float32)

    @pl.when((a == 1) & (b == 0))
    def _init():
        o_ref[...] = part

    @pl.when((a == 1) & (b == 1))
    def _acc():
        o_ref[...] = o_ref[...] + part


def _bin_extents(region):
    k = _K
    xh = xw = 1024
    i, j, h, w = region[0], region[1], region[2], region[3]
    i0, j0 = i - h / 2, j - w / 2
    i1, j1 = i + h / 2, j + w / 2
    ic = jnp.linspace(i0, i1, k + 2)[1:-1]
    jc = jnp.linspace(j0, j1, k + 2)[1:-1]
    bh, bw = h / k, w / k
    ri0 = jnp.floor((ic - bh / 2) * xh).astype(jnp.int32)
    ri1 = jnp.ceil((ic + bh / 2) * xh).astype(jnp.int32)
    rj0 = jnp.floor((jc - bw / 2) * xw).astype(jnp.int32)
    rj1 = jnp.ceil((jc + bw / 2) * xw).astype(jnp.int32)
    return ri0, ri1, rj0, rj1


def kernel(x, region):
    k2 = _K * _K
    ri0, ri1, rj0, rj1 = _bin_extents(region)

    rb0 = jnp.clip(ri0 // _B, 0, _NBLK - 1)
    cb0 = jnp.clip(rj0 // _B, 0, _NBLK - 1)
    sp = jnp.stack([rb0, cb0, ri0, ri1, rj0, rj1]).astype(jnp.int32)

    sums = pl.pallas_call(
        _psroi_kernel,
        out_shape=jax.ShapeDtypeStruct((k2, 8, 128), jnp.float32),
        grid_spec=pltpu.PrefetchScalarGridSpec(
            num_scalar_prefetch=1,
            grid=(k2, 2, 2),
            in_specs=[
                pl.BlockSpec(
                    (1, _B, _B),
                    lambda c, b, a, sp_ref: (
                        c,
                        jnp.minimum(sp_ref[0, c // _K] + a, _NBLK - 1),
                        jnp.minimum(sp_ref[1, c % _K] + b, _NBLK - 1),
                    ),
                )
            ],
            out_specs=pl.BlockSpec((1, 8, 128), lambda c, b, a, sp_ref: (c, 0, 0)),
            scratch_shapes=[pltpu.VMEM((8, 128), jnp.float32)],
        ),
        compiler_params=pltpu.CompilerParams(
            dimension_semantics=("parallel", "arbitrary", "arbitrary"),
        ),
        name="psroipool",
    )(sp, x)

    bin_sums = sums[:, 0, 0].reshape(_K, _K)
    cnt_i = (jnp.minimum(ri1, 1024) - jnp.maximum(ri0, 0)).clip(0).astype(x.dtype)
    cnt_j = (jnp.minimum(rj1, 1024) - jnp.maximum(rj0, 0)).clip(0).astype(x.dtype)
    counts = cnt_i[:, None] * cnt_j[None, :]
    return (bin_sums / counts).mean()


# speedup vs baseline: 4.0028x; 4.0028x over previous
import jax
import jax.numpy as jnp
from jax import lax
from jax.experimental import pallas as pl
from jax.experimental.pallas import tpu as pltpu

_K = 7
_BR = 96
_NRB = (1024 + _BR - 1) // _BR - 1


def _psroi_kernel(sp_ref, x_ref, o_ref, acc_ref):
    i = pl.program_id(0)
    a = pl.program_id(1)

    rb = jnp.minimum(sp_ref[0, i] + a, _NRB)
    ri0, ri1 = sp_ref[2, i], sp_ref[3, i]

    rows = rb * _BR + lax.broadcasted_iota(jnp.int32, (_BR, 1024), 0)
    row_ok = (rows >= ri0) & (rows < ri1) & ((a == 0) | (sp_ref[0, i] + a <= _NRB))

    xb = x_ref[...].astype(jnp.bfloat16).astype(jnp.float32)
    colpart = jnp.sum(jnp.where(row_ok[None], xb, 0.0), axis=1)

    @pl.when(a == 0)
    def _stash():
        acc_ref[0:_K, :] = colpart

    @pl.when(a == 1)
    def _finish():
        total = acc_ref[0:_K, :] + colpart
        q = total.astype(jnp.bfloat16).astype(jnp.float32)
        cols = lax.broadcasted_iota(jnp.int32, (1, 1024), 1)
        lane = lax.broadcasted_iota(jnp.int32, (1, 8, 128), 2)
        vals = jnp.zeros((1, 8, 128), dtype=jnp.float32)
        for j in range(_K):
            cmask = (cols >= sp_ref[4, j]) & (cols < sp_ref[5, j])
            sj = jnp.sum(jnp.where(cmask, q[j : j + 1, :], 0.0))
            vals = jnp.where(lane == j, sj, vals)
        o_ref[...] = vals


def _bin_extents(region):
    k = _K
    xh = xw = 1024
    i, j, h, w = region[0], region[1], region[2], region[3]
    i0, j0 = i - h / 2, j - w / 2
    i1, j1 = i + h / 2, j + w / 2
    ic = jnp.linspace(i0, i1, k + 2)[1:-1]
    jc = jnp.linspace(j0, j1, k + 2)[1:-1]
    bh, bw = h / k, w / k
    ri0 = jnp.floor((ic - bh / 2) * xh).astype(jnp.int32)
    ri1 = jnp.ceil((ic + bh / 2) * xh).astype(jnp.int32)
    rj0 = jnp.floor((jc - bw / 2) * xw).astype(jnp.int32)
    rj1 = jnp.ceil((jc + bw / 2) * xw).astype(jnp.int32)
    return ri0, ri1, rj0, rj1


def kernel(x, region):
    ri0, ri1, rj0, rj1 = _bin_extents(region)

    rb0 = jnp.clip(ri0 // _BR, 0, _NRB)
    sp = jnp.stack([rb0, rb0, ri0, ri1, rj0, rj1]).astype(jnp.int32)

    out = pl.pallas_call(
        _psroi_kernel,
        out_shape=jax.ShapeDtypeStruct((_K, 8, 128), jnp.float32),
        grid_spec=pltpu.PrefetchScalarGridSpec(
            num_scalar_prefetch=1,
            grid=(_K, 2),
            in_specs=[
                pl.BlockSpec(
                    (_K, _BR, 1024),
                    lambda i, a, sp_ref: (
                        i,
                        jnp.minimum(sp_ref[0, i] + a, _NRB),
                        0,
                    ),
                )
            ],
            out_specs=pl.BlockSpec((1, 8, 128), lambda i, a, sp_ref: (i, 0, 0)),
            scratch_shapes=[pltpu.VMEM((8, 1024), jnp.float32)],
        ),
        compiler_params=pltpu.CompilerParams(
            dimension_semantics=("parallel", "arbitrary"),
        ),
        name="psroipool",
    )(sp, x)

    bin_sums = out[:, 0, :_K]
    cnt_i = (jnp.minimum(ri1, 1024) - jnp.maximum(ri0, 0)).clip(0).astype(x.dtype)
    cnt_j = (jnp.minimum(rj1, 1024) - jnp.maximum(rj0, 0)).clip(0).astype(x.dtype)
    counts = cnt_i[:, None] * cnt_j[None, :]
    return (bin_sums / counts).mean()
